# baseline (speedup 1.0000x reference)
import jax
import jax.numpy as jnp
from jax.experimental import pallas as pl
from jax.experimental.pallas import tpu as pltpu

NUM_EXPERTS = 16
TOP_K = 8
N_EMBD = 768
D_FFN = 384
BLK_T = 256
N_GROUPS = 4
GROUP_E = NUM_EXPERTS // N_GROUPS
GROUP_F = GROUP_E * D_FFN


def _moe_kernel(x_ref, rw_ref, w1_ref, w2_ref, out_ref, comb_ref, acc_ref):
    g = pl.program_id(0)
    t = pl.program_id(1)
    xb = x_ref[...]

    @pl.when(g == 0)
    def _routing():
        logits = jax.lax.dot_general(
            xb, rw_ref[...], (((1,), (1,)), ((), ())),
            preferred_element_type=jnp.float32)
        col = jax.lax.broadcasted_iota(jnp.int32, logits.shape, 1)
        rank = jnp.zeros(logits.shape, dtype=jnp.int32)
        for j in range(NUM_EXPERTS):
            lj = logits[:, j:j + 1]
            beats = (lj > logits) | ((lj == logits) & (col > j))
            rank = rank + beats.astype(jnp.int32)
        sel = rank < TOP_K
        m = jnp.max(logits, axis=-1, keepdims=True)
        ew = jnp.where(sel, jnp.exp(logits - m), 0.0)
        comb_ref[t] = ew / jnp.sum(ew, axis=-1, keepdims=True)

    comb = comb_ref[t]
    col = jax.lax.broadcasted_iota(jnp.int32, comb.shape, 1)

    h = jax.lax.dot_general(
        xb, w1_ref[...], (((1,), (0,)), ((), ())),
        preferred_element_type=jnp.float32)
    h = jax.nn.gelu(h)
    parts = []
    for le in range(GROUP_E):
        e = g * GROUP_E + le
        ce = jnp.sum(jnp.where(col == e, comb, 0.0), axis=1, keepdims=True)
        parts.append(h[:, le * D_FFN:(le + 1) * D_FFN] * ce)
    gmat = jnp.concatenate(parts, axis=1)
    y = jax.lax.dot_general(
        gmat, w2_ref[...], (((1,), (0,)), ((), ())),
        preferred_element_type=jnp.float32)

    @pl.when(g == 0)
    def _init():
        acc_ref[t] = y

    @pl.when(g > 0)
    def _accum():
        acc_ref[t] += y

    @pl.when(g == N_GROUPS - 1)
    def _finish():
        out_ref[...] = acc_ref[t]


def kernel(x, router_w, w1, w2):
    B, S, D = x.shape
    T = B * S
    xt = x.reshape(T, D)
    n_t = T // BLK_T
    out = pl.pallas_call(
        _moe_kernel,
        grid=(N_GROUPS, n_t),
        in_specs=[
            pl.BlockSpec((BLK_T, D), lambda g, t: (t, 0)),
            pl.BlockSpec((NUM_EXPERTS, D), lambda g, t: (0, 0)),
            pl.BlockSpec((D, GROUP_F), lambda g, t: (0, g)),
            pl.BlockSpec((GROUP_F, D), lambda g, t: (g, 0)),
        ],
        out_specs=pl.BlockSpec((BLK_T, D), lambda g, t: (t, 0)),
        out_shape=jax.ShapeDtypeStruct((T, D), jnp.float32),
        scratch_shapes=[
            pltpu.VMEM((n_t, BLK_T, NUM_EXPERTS), jnp.float32),
            pltpu.VMEM((n_t, BLK_T, D), jnp.float32),
        ],
        compiler_params=pltpu.CompilerParams(
            dimension_semantics=("arbitrary", "arbitrary"),
        ),
    )(xt, router_w, w1, w2)
    return out.reshape(B, S, D)
